# baseline (speedup 1.0000x reference)
import functools

import jax
import jax.numpy as jnp
from jax.experimental import pallas as pl
from jax.experimental.pallas import tpu as pltpu

MAX_LEN = 512
INFLIGHT = 8


def _dma_kernel(t8_ref, out_ref, sem, *, seq_len_q, seq_len_k, max_len, inflight):
    def mk(i):
        s = (max_len - 1) - i
        c = jax.lax.rem(s, 8)
        aligned = pl.multiple_of(s - c, 8)
        return pltpu.make_async_copy(
            t8_ref.at[c, pl.ds(aligned, seq_len_k), :],
            out_ref.at[i],
            sem,
        )

    def body(i, carry):
        mk(i).start()

        @pl.when(i >= inflight)
        def _():
            mk(i - inflight).wait()

        return carry

    jax.lax.fori_loop(0, seq_len_q, body, 0)

    def tail(i, carry):
        mk(seq_len_q - inflight + i).wait()
        return carry

    jax.lax.fori_loop(0, inflight, tail, 0)


def kernel(q, k, rel_pos_enc):
    seq_len_q = q.shape[1]
    seq_len_k = k.shape[1]
    d = rel_pos_enc.shape[1]
    n = rel_pos_enc.shape[0]

    n_pad = ((n + 7) // 8) * 8 + 8
    padded = jnp.pad(rel_pos_enc, ((0, n_pad + 8 - n), (0, 0)))
    t8 = jnp.stack([jax.lax.dynamic_slice_in_dim(padded, c, n_pad, 0)
                    for c in range(8)])

    body = functools.partial(
        _dma_kernel,
        seq_len_q=seq_len_q,
        seq_len_k=seq_len_k,
        max_len=MAX_LEN,
        inflight=INFLIGHT,
    )
    return pl.pallas_call(
        body,
        in_specs=[
            pl.BlockSpec(memory_space=pltpu.MemorySpace.VMEM),
        ],
        out_specs=pl.BlockSpec(memory_space=pltpu.MemorySpace.HBM),
        out_shape=jax.ShapeDtypeStruct((seq_len_q, seq_len_k, d), rel_pos_enc.dtype),
        scratch_shapes=[pltpu.SemaphoreType.DMA],
    )(t8)
